# Initial kernel scaffold; baseline (speedup 1.0000x reference)
import jax
import jax.numpy as jnp
from jax.experimental import pallas as pl


def kernel(x_user, x_item, edge_index_clicks, edge_index_clicked_by, W1_clicks, b1_clicks, W1_clicked_by, b1_clicked_by, W2_clicks, b2_clicks, W2_clicked_by, b2_clicked_by):
    raise NotImplementedError("write your pallas kernel here")


# speedup vs baseline: 2.9887x; 2.9887x over previous
import functools

import jax
import jax.numpy as jnp
from jax import lax
from jax.experimental import pallas as pl
from jax.experimental.pallas import tpu as pltpu
from jax.experimental.pallas import tpu_sc as plsc

N = 10000
E = 320000
D = 128
NC = 2
NS = 16
EPT = E // NS
CHUNK = 80
NCHUNK = EPT // CHUNK
NP = 10240
RPT = NP // NS

_MESH = plsc.VectorSubcoreMesh(
    core_axis_name="c", subcore_axis_name="s", num_cores=NC, num_subcores=NS)


def _hist_body(idx_hbm, out_hbm, ones_hbm, zeros_hbm, idx_v, ones_v, acc, sid):
    rows = pl.ds(sid * RPT, RPT)
    pltpu.sync_copy(zeros_hbm, acc.at[rows])
    pltpu.sync_copy(ones_hbm, ones_v)
    plsc.subcore_barrier()

    def step(i, carry):
        base = sid * EPT + i * CHUNK
        pltpu.sync_copy(idx_hbm.at[pl.ds(base, CHUNK)], idx_v)
        pltpu.sync_copy(ones_v, acc.at[idx_v], add=True)
        return carry

    lax.fori_loop(0, NCHUNK, step, 0)
    plsc.subcore_barrier()
    pltpu.sync_copy(acc.at[rows], out_hbm.at[rows])


@functools.partial(
    pl.kernel,
    out_type=[jax.ShapeDtypeStruct((NP, D), jnp.float32)] * 2,
    mesh=_MESH,
    scratch_types=[
        pltpu.VMEM((CHUNK,), jnp.int32),
        pltpu.VMEM((CHUNK, D), jnp.float32),
        pltpu.VMEM_SHARED((NP, D), jnp.float32),
    ],
)
def _hist_kernel(idx_a, idx_b, ones_hbm, zeros_hbm, cnt_a, cnt_b,
                 idx_v, ones_v, acc):
    cid = lax.axis_index("c")
    sid = lax.axis_index("s")

    @pl.when(cid == 0)
    def _():
        _hist_body(idx_a, cnt_a, ones_hbm, zeros_hbm, idx_v, ones_v, acc, sid)

    @pl.when(cid == 1)
    def _():
        _hist_body(idx_b, cnt_b, ones_hbm, zeros_hbm, idx_v, ones_v, acc, sid)


def _agg_body(src_hbm, dst_hbm, feat_hbm, out_hbm, zeros_hbm,
              idx_s, idx_d, rows_v, acc, sem, sid):
    rows = pl.ds(sid * RPT, RPT)
    pltpu.sync_copy(zeros_hbm, acc.at[rows])
    plsc.subcore_barrier()

    def step(i, carry):
        base = sid * EPT + i * CHUNK
        pltpu.sync_copy(src_hbm.at[pl.ds(base, CHUNK)], idx_s)
        pltpu.sync_copy(dst_hbm.at[pl.ds(base, CHUNK)], idx_d)
        pltpu.async_copy(feat_hbm.at[idx_s], rows_v, sem).wait()
        pltpu.sync_copy(rows_v, acc.at[idx_d], add=True)
        return carry

    lax.fori_loop(0, NCHUNK, step, 0)
    plsc.subcore_barrier()
    pltpu.sync_copy(acc.at[rows], out_hbm.at[rows])


@functools.partial(
    pl.kernel,
    out_type=[jax.ShapeDtypeStruct((NP, D), jnp.float32)] * 2,
    mesh=_MESH,
    scratch_types=[
        pltpu.VMEM((CHUNK,), jnp.int32),
        pltpu.VMEM((CHUNK,), jnp.int32),
        pltpu.VMEM((CHUNK, D), jnp.float32),
        pltpu.VMEM_SHARED((NP, D), jnp.float32),
        pltpu.SemaphoreType.DMA,
    ],
)
def _agg_kernel(src_c, dst_c, src_b, dst_b, xu_s, xi_s, zeros_hbm,
                agg_item, agg_user,
                idx_s, idx_d, rows_v, acc, sem):
    cid = lax.axis_index("c")
    sid = lax.axis_index("s")

    @pl.when(cid == 0)
    def _():
        _agg_body(src_c, dst_c, xu_s, agg_item, zeros_hbm,
                  idx_s, idx_d, rows_v, acc, sem, sid)

    @pl.when(cid == 1)
    def _():
        _agg_body(src_b, dst_b, xi_s, agg_user, zeros_hbm,
                  idx_s, idx_d, rows_v, acc, sem, sid)


_BLK = 1000
_GRID = N // _BLK


def _rs(cnt_ref):
    return lax.rsqrt(jnp.maximum(cnt_ref[:, 0:1], 1.0))


def _prescale_body(xu_ref, cu_ref, xi_ref, ci_ref, ou_ref, oi_ref):
    ou_ref[...] = xu_ref[...] * _rs(cu_ref)
    oi_ref[...] = xi_ref[...] * _rs(ci_ref)


def _layer1_body(ai_ref, cii_ref, cio_ref, wc_ref, bc_ref,
                 au_ref, cui_ref, cuo_ref, wb_ref, bb_ref,
                 oi_ref, ou_ref):
    hi = (ai_ref[...] * _rs(cii_ref)) @ wc_ref[...] + bc_ref[...]
    oi_ref[...] = jnp.maximum(hi, 0.0) * _rs(cio_ref)
    hu = (au_ref[...] * _rs(cui_ref)) @ wb_ref[...] + bb_ref[...]
    ou_ref[...] = jnp.maximum(hu, 0.0) * _rs(cuo_ref)


def _layer2_body(ai_ref, cii_ref, wc_ref, bc_ref,
                 au_ref, cui_ref, wb_ref, bb_ref,
                 oi_ref, ou_ref):
    oi_ref[...] = (ai_ref[...] * _rs(cii_ref)) @ wc_ref[...] + bc_ref[...]
    ou_ref[...] = (au_ref[...] * _rs(cui_ref)) @ wb_ref[...] + bb_ref[...]


def _row_spec(width):
    return pl.BlockSpec((_BLK, width), lambda i: (i, 0))


_FULL_W = pl.BlockSpec((D, D), lambda i: (0, 0))
_FULL_B = pl.BlockSpec((1, D), lambda i: (0, 0))
_FEAT = functools.partial(jax.ShapeDtypeStruct, (N, D))


def _prescale(xu, cu, xi, ci):
    return pl.pallas_call(
        _prescale_body,
        grid=(_GRID,),
        in_specs=[_row_spec(D), _row_spec(D), _row_spec(D), _row_spec(D)],
        out_specs=[_row_spec(D), _row_spec(D)],
        out_shape=[_FEAT(jnp.float32)] * 2,
    )(xu, cu, xi, ci)


def _layer1(ai, cii, cio, wc, bc, au, cui, cuo, wb, bb):
    return pl.pallas_call(
        _layer1_body,
        grid=(_GRID,),
        in_specs=[_row_spec(D), _row_spec(D), _row_spec(D), _FULL_W, _FULL_B,
                  _row_spec(D), _row_spec(D), _row_spec(D), _FULL_W, _FULL_B],
        out_specs=[_row_spec(D), _row_spec(D)],
        out_shape=[_FEAT(jnp.float32)] * 2,
    )(ai, cii, cio, wc, bc, au, cui, cuo, wb, bb)


def _layer2(ai, cii, wc, bc, au, cui, wb, bb):
    return pl.pallas_call(
        _layer2_body,
        grid=(_GRID,),
        in_specs=[_row_spec(D), _row_spec(D), _FULL_W, _FULL_B,
                  _row_spec(D), _row_spec(D), _FULL_W, _FULL_B],
        out_specs=[_row_spec(D), _row_spec(D)],
        out_shape=[_FEAT(jnp.float32)] * 2,
    )(ai, cii, wc, bc, au, cui, wb, bb)


@jax.jit
def kernel(x_user, x_item, edge_index_clicks, edge_index_clicked_by,
           W1_clicks, b1_clicks, W1_clicked_by, b1_clicked_by,
           W2_clicks, b2_clicks, W2_clicked_by, b2_clicked_by):
    src_c = edge_index_clicks[0].astype(jnp.int32)
    dst_c = edge_index_clicks[1].astype(jnp.int32)
    src_b = edge_index_clicked_by[0].astype(jnp.int32)
    dst_b = edge_index_clicked_by[1].astype(jnp.int32)

    zeros_feat = jnp.zeros((RPT, D), jnp.float32)
    ones_feat = jnp.ones((CHUNK, D), jnp.float32)

    cnt_sc, cnt_sb = _hist_kernel(src_c, src_b, ones_feat, zeros_feat)
    cnt_dc, cnt_db = _hist_kernel(dst_c, dst_b, ones_feat, zeros_feat)
    cnt_sc, cnt_dc = cnt_sc[:N], cnt_dc[:N]
    cnt_sb, cnt_db = cnt_sb[:N], cnt_db[:N]

    b1c = b1_clicks.reshape(1, D)
    b1b = b1_clicked_by.reshape(1, D)
    b2c = b2_clicks.reshape(1, D)
    b2b = b2_clicked_by.reshape(1, D)

    xu_s, xi_s = _prescale(x_user, cnt_sc, x_item, cnt_sb)
    agg_item, agg_user = _agg_kernel(
        src_c, dst_c, src_b, dst_b, xu_s, xi_s, zeros_feat)
    agg_item, agg_user = agg_item[:N], agg_user[:N]
    hi_s, hu_s = _layer1(agg_item, cnt_dc, cnt_sb, W1_clicks, b1c,
                         agg_user, cnt_db, cnt_sc, W1_clicked_by, b1b)

    agg2_item, agg2_user = _agg_kernel(
        src_c, dst_c, src_b, dst_b, hu_s, hi_s, zeros_feat)
    agg2_item, agg2_user = agg2_item[:N], agg2_user[:N]
    out_item, out_user = _layer2(agg2_item, cnt_dc, W2_clicks, b2c,
                                 agg2_user, cnt_db, W2_clicked_by, b2b)
    return (out_user, out_item)


# speedup vs baseline: 4.1782x; 1.3980x over previous
import functools

import jax
import jax.numpy as jnp
from jax import lax
from jax.experimental import pallas as pl
from jax.experimental.pallas import tpu as pltpu
from jax.experimental.pallas import tpu_sc as plsc

N = 10000
E = 320000
D = 128
NC = 2
NS = 16
EPT = E // NS
CHUNK = 80
NCHUNK = EPT // CHUNK
NP = 10240
RPT = NP // NS

_MESH = plsc.VectorSubcoreMesh(
    core_axis_name="c", subcore_axis_name="s", num_cores=NC, num_subcores=NS)


def _hist_body(idx_hbm, out_hbm, ones_hbm, zeros_hbm, idx_v, ones_v, acc, sid):
    rows = pl.ds(sid * RPT, RPT)
    pltpu.sync_copy(zeros_hbm, acc.at[rows])
    pltpu.sync_copy(ones_hbm, ones_v)
    plsc.subcore_barrier()

    def step(i, carry):
        base = sid * EPT + i * CHUNK
        pltpu.sync_copy(idx_hbm.at[pl.ds(base, CHUNK)], idx_v)
        pltpu.sync_copy(ones_v, acc.at[idx_v], add=True)
        return carry

    lax.fori_loop(0, NCHUNK, step, 0)
    plsc.subcore_barrier()
    pltpu.sync_copy(acc.at[rows], out_hbm.at[rows])


@functools.partial(
    pl.kernel,
    out_type=[jax.ShapeDtypeStruct((NP, D), jnp.float32)] * 2,
    mesh=_MESH,
    scratch_types=[
        pltpu.VMEM((CHUNK,), jnp.int32),
        pltpu.VMEM((CHUNK, D), jnp.float32),
        pltpu.VMEM_SHARED((NP, D), jnp.float32),
    ],
)
def _hist_kernel(idx_a, idx_b, ones_hbm, zeros_hbm, cnt_a, cnt_b,
                 idx_v, ones_v, acc):
    cid = lax.axis_index("c")
    sid = lax.axis_index("s")

    @pl.when(cid == 0)
    def _():
        _hist_body(idx_a, cnt_a, ones_hbm, zeros_hbm, idx_v, ones_v, acc, sid)

    @pl.when(cid == 1)
    def _():
        _hist_body(idx_b, cnt_b, ones_hbm, zeros_hbm, idx_v, ones_v, acc, sid)


NPAIR = NCHUNK // 2


def _agg_body(src_hbm, dst_hbm, feat_hbm, out_hbm, zeros_hbm,
              idx_sa, idx_da, idx_sb, idx_db, rows_a, rows_b,
              acc, sem_a, sem_b, sid):
    rows = pl.ds(sid * RPT, RPT)
    pltpu.sync_copy(zeros_hbm, acc.at[rows])
    plsc.subcore_barrier()

    ebase = sid * EPT

    def load_idx(i, idx_s, idx_d):
        base = ebase + i * CHUNK
        pltpu.sync_copy(src_hbm.at[pl.ds(base, CHUNK)], idx_s)
        pltpu.sync_copy(dst_hbm.at[pl.ds(base, CHUNK)], idx_d)

    load_idx(0, idx_sa, idx_da)
    ga = pltpu.async_copy(feat_hbm.at[idx_sa], rows_a, sem_a)

    def pair(k, carry):
        load_idx(2 * k + 1, idx_sb, idx_db)
        gb = pltpu.async_copy(feat_hbm.at[idx_sb], rows_b, sem_b)
        pltpu.make_async_copy(feat_hbm.at[idx_sa], rows_a, sem_a).wait()
        pltpu.sync_copy(rows_a, acc.at[idx_da], add=True)

        @pl.when(k < NPAIR - 1)
        def _():
            load_idx(2 * k + 2, idx_sa, idx_da)
            pltpu.async_copy(feat_hbm.at[idx_sa], rows_a, sem_a)

        pltpu.make_async_copy(feat_hbm.at[idx_sb], rows_b, sem_b).wait()
        pltpu.sync_copy(rows_b, acc.at[idx_db], add=True)
        return carry

    lax.fori_loop(0, NPAIR, pair, 0)
    plsc.subcore_barrier()
    pltpu.sync_copy(acc.at[rows], out_hbm.at[rows])


@functools.partial(
    pl.kernel,
    out_type=[jax.ShapeDtypeStruct((NP, D), jnp.float32)] * 2,
    mesh=_MESH,
    scratch_types=[
        pltpu.VMEM((CHUNK,), jnp.int32),
        pltpu.VMEM((CHUNK,), jnp.int32),
        pltpu.VMEM((CHUNK,), jnp.int32),
        pltpu.VMEM((CHUNK,), jnp.int32),
        pltpu.VMEM((CHUNK, D), jnp.float32),
        pltpu.VMEM((CHUNK, D), jnp.float32),
        pltpu.VMEM_SHARED((NP, D), jnp.float32),
        pltpu.SemaphoreType.DMA,
        pltpu.SemaphoreType.DMA,
    ],
)
def _agg_kernel(src_c, dst_c, src_b, dst_b, xu_s, xi_s, zeros_hbm,
                agg_item, agg_user,
                idx_sa, idx_da, idx_sb, idx_db, rows_a, rows_b,
                acc, sem_a, sem_b):
    cid = lax.axis_index("c")
    sid = lax.axis_index("s")

    @pl.when(cid == 0)
    def _():
        _agg_body(src_c, dst_c, xu_s, agg_item, zeros_hbm,
                  idx_sa, idx_da, idx_sb, idx_db, rows_a, rows_b,
                  acc, sem_a, sem_b, sid)

    @pl.when(cid == 1)
    def _():
        _agg_body(src_b, dst_b, xi_s, agg_user, zeros_hbm,
                  idx_sa, idx_da, idx_sb, idx_db, rows_a, rows_b,
                  acc, sem_a, sem_b, sid)


_BLK = 1000
_GRID = N // _BLK


def _rs(cnt_ref):
    return lax.rsqrt(jnp.maximum(cnt_ref[:, 0:1], 1.0))


def _prescale_body(xu_ref, cu_ref, xi_ref, ci_ref, ou_ref, oi_ref):
    ou_ref[...] = xu_ref[...] * _rs(cu_ref)
    oi_ref[...] = xi_ref[...] * _rs(ci_ref)


def _layer1_body(ai_ref, cii_ref, cio_ref, wc_ref, bc_ref,
                 au_ref, cui_ref, cuo_ref, wb_ref, bb_ref,
                 oi_ref, ou_ref):
    hi = (ai_ref[...] * _rs(cii_ref)) @ wc_ref[...] + bc_ref[...]
    oi_ref[...] = jnp.maximum(hi, 0.0) * _rs(cio_ref)
    hu = (au_ref[...] * _rs(cui_ref)) @ wb_ref[...] + bb_ref[...]
    ou_ref[...] = jnp.maximum(hu, 0.0) * _rs(cuo_ref)


def _layer2_body(ai_ref, cii_ref, wc_ref, bc_ref,
                 au_ref, cui_ref, wb_ref, bb_ref,
                 oi_ref, ou_ref):
    oi_ref[...] = (ai_ref[...] * _rs(cii_ref)) @ wc_ref[...] + bc_ref[...]
    ou_ref[...] = (au_ref[...] * _rs(cui_ref)) @ wb_ref[...] + bb_ref[...]


def _row_spec(width):
    return pl.BlockSpec((_BLK, width), lambda i: (i, 0))


_FULL_W = pl.BlockSpec((D, D), lambda i: (0, 0))
_FULL_B = pl.BlockSpec((1, D), lambda i: (0, 0))
_FEAT = functools.partial(jax.ShapeDtypeStruct, (N, D))


def _prescale(xu, cu, xi, ci):
    return pl.pallas_call(
        _prescale_body,
        grid=(_GRID,),
        in_specs=[_row_spec(D), _row_spec(D), _row_spec(D), _row_spec(D)],
        out_specs=[_row_spec(D), _row_spec(D)],
        out_shape=[_FEAT(jnp.float32)] * 2,
    )(xu, cu, xi, ci)


def _layer1(ai, cii, cio, wc, bc, au, cui, cuo, wb, bb):
    return pl.pallas_call(
        _layer1_body,
        grid=(_GRID,),
        in_specs=[_row_spec(D), _row_spec(D), _row_spec(D), _FULL_W, _FULL_B,
                  _row_spec(D), _row_spec(D), _row_spec(D), _FULL_W, _FULL_B],
        out_specs=[_row_spec(D), _row_spec(D)],
        out_shape=[_FEAT(jnp.float32)] * 2,
    )(ai, cii, cio, wc, bc, au, cui, cuo, wb, bb)


def _layer2(ai, cii, wc, bc, au, cui, wb, bb):
    return pl.pallas_call(
        _layer2_body,
        grid=(_GRID,),
        in_specs=[_row_spec(D), _row_spec(D), _FULL_W, _FULL_B,
                  _row_spec(D), _row_spec(D), _FULL_W, _FULL_B],
        out_specs=[_row_spec(D), _row_spec(D)],
        out_shape=[_FEAT(jnp.float32)] * 2,
    )(ai, cii, wc, bc, au, cui, wb, bb)


@jax.jit
def kernel(x_user, x_item, edge_index_clicks, edge_index_clicked_by,
           W1_clicks, b1_clicks, W1_clicked_by, b1_clicked_by,
           W2_clicks, b2_clicks, W2_clicked_by, b2_clicked_by):
    src_c = edge_index_clicks[0].astype(jnp.int32)
    dst_c = edge_index_clicks[1].astype(jnp.int32)
    src_b = edge_index_clicked_by[0].astype(jnp.int32)
    dst_b = edge_index_clicked_by[1].astype(jnp.int32)

    zeros_feat = jnp.zeros((RPT, D), jnp.float32)
    ones_feat = jnp.ones((CHUNK, D), jnp.float32)

    cnt_sc, cnt_sb = _hist_kernel(src_c, src_b, ones_feat, zeros_feat)
    cnt_dc, cnt_db = _hist_kernel(dst_c, dst_b, ones_feat, zeros_feat)
    cnt_sc, cnt_dc = cnt_sc[:N], cnt_dc[:N]
    cnt_sb, cnt_db = cnt_sb[:N], cnt_db[:N]

    b1c = b1_clicks.reshape(1, D)
    b1b = b1_clicked_by.reshape(1, D)
    b2c = b2_clicks.reshape(1, D)
    b2b = b2_clicked_by.reshape(1, D)

    xu_s, xi_s = _prescale(x_user, cnt_sc, x_item, cnt_sb)
    agg_item, agg_user = _agg_kernel(
        src_c, dst_c, src_b, dst_b, xu_s, xi_s, zeros_feat)
    agg_item, agg_user = agg_item[:N], agg_user[:N]
    hi_s, hu_s = _layer1(agg_item, cnt_dc, cnt_sb, W1_clicks, b1c,
                         agg_user, cnt_db, cnt_sc, W1_clicked_by, b1b)

    agg2_item, agg2_user = _agg_kernel(
        src_c, dst_c, src_b, dst_b, hu_s, hi_s, zeros_feat)
    agg2_item, agg2_user = agg2_item[:N], agg2_user[:N]
    out_item, out_user = _layer2(agg2_item, cnt_dc, W2_clicks, b2c,
                                 agg2_user, cnt_db, W2_clicked_by, b2b)
    return (out_user, out_item)


# speedup vs baseline: 4.9858x; 1.1933x over previous
import functools

import jax
import jax.numpy as jnp
from jax import lax
from jax.experimental import pallas as pl
from jax.experimental.pallas import tpu as pltpu
from jax.experimental.pallas import tpu_sc as plsc

N = 10000
E = 320000
D = 128
NC = 2
NS = 16
EPT = E // NS
CHUNK = 80
NCHUNK = EPT // CHUNK
NP = 10240
RPT = NP // NS

_MESH = plsc.VectorSubcoreMesh(
    core_axis_name="c", subcore_axis_name="s", num_cores=NC, num_subcores=NS)


def _hist_body(idx_hbm, out_hbm, ones_hbm, zeros_hbm,
               idx_va, idx_vb, ones_v, acc, isem_a, isem_b, sid):
    rows = pl.ds(sid * RPT, RPT)
    pltpu.sync_copy(zeros_hbm, acc.at[rows])
    pltpu.sync_copy(ones_hbm, ones_v)
    plsc.subcore_barrier()

    ebase = sid * EPT

    def start_idx(i, idx_v, isem):
        pltpu.async_copy(idx_hbm.at[pl.ds(ebase + i * CHUNK, CHUNK)],
                         idx_v, isem)

    start_idx(0, idx_va, isem_a)

    def pair(k, carry):
        start_idx(2 * k + 1, idx_vb, isem_b)
        pltpu.make_async_copy(idx_hbm.at[pl.ds(0, CHUNK)], idx_va,
                              isem_a).wait()
        pltpu.sync_copy(ones_v, acc.at[idx_va], add=True)

        @pl.when(k < NPAIR - 1)
        def _():
            start_idx(2 * k + 2, idx_va, isem_a)

        pltpu.make_async_copy(idx_hbm.at[pl.ds(0, CHUNK)], idx_vb,
                              isem_b).wait()
        pltpu.sync_copy(ones_v, acc.at[idx_vb], add=True)
        return carry

    lax.fori_loop(0, NPAIR, pair, 0)
    plsc.subcore_barrier()
    pltpu.sync_copy(acc.at[rows], out_hbm.at[rows])


@functools.partial(
    pl.kernel,
    out_type=[jax.ShapeDtypeStruct((NP, D), jnp.float32)] * 2,
    mesh=_MESH,
    scratch_types=[
        pltpu.VMEM((CHUNK,), jnp.int32),
        pltpu.VMEM((CHUNK,), jnp.int32),
        pltpu.VMEM((CHUNK, D), jnp.float32),
        pltpu.VMEM_SHARED((NP, D), jnp.float32),
        pltpu.SemaphoreType.DMA,
        pltpu.SemaphoreType.DMA,
    ],
)
def _hist_kernel(idx_a, idx_b, ones_hbm, zeros_hbm, cnt_a, cnt_b,
                 idx_va, idx_vb, ones_v, acc, isem_a, isem_b):
    cid = lax.axis_index("c")
    sid = lax.axis_index("s")

    @pl.when(cid == 0)
    def _():
        _hist_body(idx_a, cnt_a, ones_hbm, zeros_hbm,
                   idx_va, idx_vb, ones_v, acc, isem_a, isem_b, sid)

    @pl.when(cid == 1)
    def _():
        _hist_body(idx_b, cnt_b, ones_hbm, zeros_hbm,
                   idx_va, idx_vb, ones_v, acc, isem_a, isem_b, sid)


NPAIR = NCHUNK // 2


def _agg_body(src_hbm, dst_hbm, feat_hbm, out_hbm, zeros_hbm,
              idx_sa, idx_da, idx_sb, idx_db, rows_a, rows_b,
              acc, sem_a, sem_b, sid):
    rows = pl.ds(sid * RPT, RPT)
    pltpu.sync_copy(zeros_hbm, acc.at[rows])
    plsc.subcore_barrier()

    ebase = sid * EPT

    def load_idx(i, idx_s, idx_d):
        base = ebase + i * CHUNK
        pltpu.sync_copy(src_hbm.at[pl.ds(base, CHUNK)], idx_s)
        pltpu.sync_copy(dst_hbm.at[pl.ds(base, CHUNK)], idx_d)

    load_idx(0, idx_sa, idx_da)
    ga = pltpu.async_copy(feat_hbm.at[idx_sa], rows_a, sem_a)

    def pair(k, carry):
        load_idx(2 * k + 1, idx_sb, idx_db)
        gb = pltpu.async_copy(feat_hbm.at[idx_sb], rows_b, sem_b)
        pltpu.make_async_copy(feat_hbm.at[idx_sa], rows_a, sem_a).wait()
        pltpu.sync_copy(rows_a, acc.at[idx_da], add=True)

        @pl.when(k < NPAIR - 1)
        def _():
            load_idx(2 * k + 2, idx_sa, idx_da)
            pltpu.async_copy(feat_hbm.at[idx_sa], rows_a, sem_a)

        pltpu.make_async_copy(feat_hbm.at[idx_sb], rows_b, sem_b).wait()
        pltpu.sync_copy(rows_b, acc.at[idx_db], add=True)
        return carry

    lax.fori_loop(0, NPAIR, pair, 0)
    plsc.subcore_barrier()
    pltpu.sync_copy(acc.at[rows], out_hbm.at[rows])


@functools.partial(
    pl.kernel,
    out_type=[jax.ShapeDtypeStruct((NP, D), jnp.float32)] * 2,
    mesh=_MESH,
    scratch_types=[
        pltpu.VMEM((CHUNK,), jnp.int32),
        pltpu.VMEM((CHUNK,), jnp.int32),
        pltpu.VMEM((CHUNK,), jnp.int32),
        pltpu.VMEM((CHUNK,), jnp.int32),
        pltpu.VMEM((CHUNK, D), jnp.float32),
        pltpu.VMEM((CHUNK, D), jnp.float32),
        pltpu.VMEM_SHARED((NP, D), jnp.float32),
        pltpu.SemaphoreType.DMA,
        pltpu.SemaphoreType.DMA,
    ],
)
def _agg_kernel(src_c, dst_c, src_b, dst_b, xu_s, xi_s, zeros_hbm,
                agg_item, agg_user,
                idx_sa, idx_da, idx_sb, idx_db, rows_a, rows_b,
                acc, sem_a, sem_b):
    cid = lax.axis_index("c")
    sid = lax.axis_index("s")

    @pl.when(cid == 0)
    def _():
        _agg_body(src_c, dst_c, xu_s, agg_item, zeros_hbm,
                  idx_sa, idx_da, idx_sb, idx_db, rows_a, rows_b,
                  acc, sem_a, sem_b, sid)

    @pl.when(cid == 1)
    def _():
        _agg_body(src_b, dst_b, xi_s, agg_user, zeros_hbm,
                  idx_sa, idx_da, idx_sb, idx_db, rows_a, rows_b,
                  acc, sem_a, sem_b, sid)


_BLK = 1000
_GRID = N // _BLK


def _rs(cnt_ref):
    return lax.rsqrt(jnp.maximum(cnt_ref[:, 0:1], 1.0))


def _prescale_body(xu_ref, cu_ref, xi_ref, ci_ref, ou_ref, oi_ref):
    ou_ref[...] = xu_ref[...] * _rs(cu_ref)
    oi_ref[...] = xi_ref[...] * _rs(ci_ref)


def _layer1_body(ai_ref, cii_ref, cio_ref, wc_ref, bc_ref,
                 au_ref, cui_ref, cuo_ref, wb_ref, bb_ref,
                 oi_ref, ou_ref):
    hi = (ai_ref[...] * _rs(cii_ref)) @ wc_ref[...] + bc_ref[...]
    oi_ref[...] = jnp.maximum(hi, 0.0) * _rs(cio_ref)
    hu = (au_ref[...] * _rs(cui_ref)) @ wb_ref[...] + bb_ref[...]
    ou_ref[...] = jnp.maximum(hu, 0.0) * _rs(cuo_ref)


def _layer2_body(ai_ref, cii_ref, wc_ref, bc_ref,
                 au_ref, cui_ref, wb_ref, bb_ref,
                 oi_ref, ou_ref):
    oi_ref[...] = (ai_ref[...] * _rs(cii_ref)) @ wc_ref[...] + bc_ref[...]
    ou_ref[...] = (au_ref[...] * _rs(cui_ref)) @ wb_ref[...] + bb_ref[...]


def _row_spec(width):
    return pl.BlockSpec((_BLK, width), lambda i: (i, 0))


_FULL_W = pl.BlockSpec((D, D), lambda i: (0, 0))
_FULL_B = pl.BlockSpec((1, D), lambda i: (0, 0))
_FEAT = functools.partial(jax.ShapeDtypeStruct, (N, D))


def _prescale(xu, cu, xi, ci):
    return pl.pallas_call(
        _prescale_body,
        grid=(_GRID,),
        in_specs=[_row_spec(D), _row_spec(D), _row_spec(D), _row_spec(D)],
        out_specs=[_row_spec(D), _row_spec(D)],
        out_shape=[_FEAT(jnp.float32)] * 2,
    )(xu, cu, xi, ci)


def _layer1(ai, cii, cio, wc, bc, au, cui, cuo, wb, bb):
    return pl.pallas_call(
        _layer1_body,
        grid=(_GRID,),
        in_specs=[_row_spec(D), _row_spec(D), _row_spec(D), _FULL_W, _FULL_B,
                  _row_spec(D), _row_spec(D), _row_spec(D), _FULL_W, _FULL_B],
        out_specs=[_row_spec(D), _row_spec(D)],
        out_shape=[_FEAT(jnp.float32)] * 2,
    )(ai, cii, cio, wc, bc, au, cui, cuo, wb, bb)


def _layer2(ai, cii, wc, bc, au, cui, wb, bb):
    return pl.pallas_call(
        _layer2_body,
        grid=(_GRID,),
        in_specs=[_row_spec(D), _row_spec(D), _FULL_W, _FULL_B,
                  _row_spec(D), _row_spec(D), _FULL_W, _FULL_B],
        out_specs=[_row_spec(D), _row_spec(D)],
        out_shape=[_FEAT(jnp.float32)] * 2,
    )(ai, cii, wc, bc, au, cui, wb, bb)


@jax.jit
def kernel(x_user, x_item, edge_index_clicks, edge_index_clicked_by,
           W1_clicks, b1_clicks, W1_clicked_by, b1_clicked_by,
           W2_clicks, b2_clicks, W2_clicked_by, b2_clicked_by):
    src_c = edge_index_clicks[0].astype(jnp.int32)
    dst_c = edge_index_clicks[1].astype(jnp.int32)
    src_b = edge_index_clicked_by[0].astype(jnp.int32)
    dst_b = edge_index_clicked_by[1].astype(jnp.int32)

    zeros_feat = jnp.zeros((RPT, D), jnp.float32)
    ones_feat = jnp.ones((CHUNK, D), jnp.float32)

    cnt_sc, cnt_sb = _hist_kernel(src_c, src_b, ones_feat, zeros_feat)
    cnt_dc, cnt_db = _hist_kernel(dst_c, dst_b, ones_feat, zeros_feat)
    cnt_sc, cnt_dc = cnt_sc[:N], cnt_dc[:N]
    cnt_sb, cnt_db = cnt_sb[:N], cnt_db[:N]

    b1c = b1_clicks.reshape(1, D)
    b1b = b1_clicked_by.reshape(1, D)
    b2c = b2_clicks.reshape(1, D)
    b2b = b2_clicked_by.reshape(1, D)

    xu_s, xi_s = _prescale(x_user, cnt_sc, x_item, cnt_sb)
    agg_item, agg_user = _agg_kernel(
        src_c, dst_c, src_b, dst_b, xu_s, xi_s, zeros_feat)
    agg_item, agg_user = agg_item[:N], agg_user[:N]
    hi_s, hu_s = _layer1(agg_item, cnt_dc, cnt_sb, W1_clicks, b1c,
                         agg_user, cnt_db, cnt_sc, W1_clicked_by, b1b)

    agg2_item, agg2_user = _agg_kernel(
        src_c, dst_c, src_b, dst_b, hu_s, hi_s, zeros_feat)
    agg2_item, agg2_user = agg2_item[:N], agg2_user[:N]
    out_item, out_user = _layer2(agg2_item, cnt_dc, W2_clicks, b2c,
                                 agg2_user, cnt_db, W2_clicked_by, b2b)
    return (out_user, out_item)
